# baseline (speedup 1.0000x reference)
import functools

import numpy as np
import jax
import jax.numpy as jnp
from jax import lax
from jax.experimental import pallas as pl
from jax.experimental.pallas import tpu as pltpu
from jax.experimental.pallas import tpu_sc as plsc

B = 64
NOBJ = 20
A = 3
NCLS = 3
GS = (76, 38, 19)
IGNORE = 0.5
NSLOT = A * B * NOBJ
NW = 32
ROWS_W = NSLOT // NW

_ANCH = (
    tuple((w / 8.0, h / 8.0) for w, h in ((10., 13.), (16., 30.), (33., 23.))),
    tuple((w / 16.0, h / 16.0) for w, h in ((30., 61.), (62., 45.), (59., 119.))),
    tuple((w / 32.0, h / 32.0) for w, h in ((116., 90.), (156., 198.), (373., 326.))),
)

_DROWS = (69312, 17328, 4332)
_BLK = 2888
_STEPS = (24, 6, 1)


def _bce0(p):
    return -jnp.maximum(jnp.log(1.0 - p), -100.0)


def _bce1(p):
    return -jnp.maximum(jnp.log(p), -100.0)


def _prep_body(boxes_ref, labels_ref, i0, p0, i1, p1, i2, p2):
    bx = boxes_ref[...]
    lbl = labels_ref[...]
    bidx = lax.broadcasted_iota(jnp.int32, (B, NOBJ), 0)
    n1 = lax.broadcasted_iota(jnp.int32, (B, NOBJ, NOBJ), 1)
    n2 = lax.broadcasted_iota(jnp.int32, (B, NOBJ, NOBJ), 2)
    later = n2 > n1
    earlier = n2 < n1
    for s, (iref, pref) in enumerate(((i0, p0), (i1, p1), (i2, p2))):
        g = float(GS[s])
        gn = GS[s]
        tx = bx[:, :, 0] * g
        ty = bx[:, :, 1] * g
        tw = bx[:, :, 2] * g
        th = bx[:, :, 3] * g
        gi = tx.astype(jnp.int32)
        gj = ty.astype(jnp.int32)
        ious = []
        for a in range(A):
            aw, ah = _ANCH[s][a]
            inter = jnp.minimum(aw, tw) * jnp.minimum(ah, th)
            union = aw * ah + 1e-16 + tw * th - inter
            ious.append(inter / union)
        b01 = jnp.where(ious[1] > ious[0], 1, 0)
        m01 = jnp.maximum(ious[0], ious[1])
        best = jnp.where(ious[2] > m01, 2, b01)
        same = (gi[:, :, None] == gi[:, None, :]) & (gj[:, :, None] == gj[:, None, :])
        dup = same & (best[:, :, None] == best[:, None, :])
        winner = jnp.logical_not(jnp.any(dup & later, axis=2))
        tcks = []
        for k in range(NCLS):
            tck = jnp.any(dup & (lbl[:, None, :] == k), axis=2)
            tcks.append(tck.astype(jnp.float32))
        for a in range(A):
            z_a = (best == a) | (ious[a] > IGNORE)
            repz = z_a & jnp.logical_not(
                jnp.any(same & z_a[:, None, :] & earlier, axis=2))
            cell = ((bidx * A + a) * gn + gj) * gn + gi
            iref[a] = cell >> 4
            pref[0, a] = repz.astype(jnp.float32)
            pref[1, a] = (winner & (best == a)).astype(jnp.float32)
            pref[2, a] = tw
            pref[3, a] = th
            for k in range(NCLS):
                pref[4 + k, a] = tcks[k]
            pref[7, a] = (cell & 15).astype(jnp.float32)


_prep = pl.pallas_call(
    _prep_body,
    out_shape=(
        jax.ShapeDtypeStruct((A, B, NOBJ), jnp.int32),
        jax.ShapeDtypeStruct((8, A, B, NOBJ), jnp.float32),
        jax.ShapeDtypeStruct((A, B, NOBJ), jnp.int32),
        jax.ShapeDtypeStruct((8, A, B, NOBJ), jnp.float32),
        jax.ShapeDtypeStruct((A, B, NOBJ), jnp.int32),
        jax.ShapeDtypeStruct((8, A, B, NOBJ), jnp.float32),
    ),
)


@functools.cache
def _make_sc_gather():
    mesh = plsc.VectorSubcoreMesh(core_axis_name="c", subcore_axis_name="s")

    @functools.partial(
        pl.kernel,
        mesh=mesh,
        out_type=[jax.ShapeDtypeStruct((NSLOT, 128), jnp.float32)] * 3,
        scratch_types=[
            pltpu.VMEM((ROWS_W,), jnp.int32),
            pltpu.VMEM((ROWS_W, 128), jnp.float32),
            pltpu.SemaphoreType.DMA,
        ],
    )
    def _sc_gather(t0, t1, t2, i0, i1, i2, o0, o1, o2, row_v, rows_v, sem):
        wid = lax.axis_index("s") * 2 + lax.axis_index("c")
        base = wid * ROWS_W
        for t, i, o in ((t0, i0, o0), (t1, i1, o1), (t2, i2, o2)):
            pltpu.sync_copy(i.at[pl.ds(base, ROWS_W)], row_v)
            pltpu.async_copy(t.at[row_v], rows_v, sem).wait()
            pltpu.sync_copy(rows_v, o.at[pl.ds(base, ROWS_W)])

    return _sc_gather


def _final_body(d0, d1, d2, pt0, pt1, pt2, pk0, pk1, pk2, out_ref):
    step = pl.program_id(0)

    @pl.when(step == 0)
    def _sparse():
        acc = jnp.float32(0.0)
        r = lax.broadcasted_iota(jnp.int32, (30, 128), 0) // 10
        for s, (pt, pk) in enumerate(((pt0, pk0), (pt1, pk1), (pt2, pk2))):
            a0, a1, a2 = _ANCH[s]
            aw = jnp.where(r == 0, a0[0], jnp.where(r == 1, a1[0], a2[0]))
            ah = jnp.where(r == 0, a0[1], jnp.where(r == 1, a1[1], a2[1]))
            ov = pk[7]
            chs = []
            for c in range(8):
                ch = jnp.float32(0.0)
                for h in range(16):
                    ch = ch + jnp.where(ov == float(h), pt[h * 8 + c], 0.0)
                chs.append(ch)
            px, py, pw, ph, pc = chs[0], chs[1], chs[2], chs[3], chs[4]
            zm = pk[0]
            om = pk[1]
            tw = pk[2]
            th = pk[3]
            fx = px - jnp.floor(px)
            fy = py - jnp.floor(py)
            dw = jnp.log(pw / aw + 1e-16) - jnp.log(tw / aw + 1e-16)
            dh = jnp.log(ph / ah + 1e-16) - jnp.log(th / ah + 1e-16)
            cls = jnp.float32(0.0)
            for k in range(NCLS):
                pkk = chs[5 + k]
                tck = pk[4 + k]
                cls = cls + tck * _bce1(pkk) + (1.0 - tck) * _bce0(pkk)
            objterm = fx * fx + fy * fy + dw * dw + dh * dh + 5.0 * _bce1(pc) + cls
            acc = acc + jnp.sum(om * objterm) - 0.5 * jnp.sum(zm * _bce0(pc))
        out_ref[0, 0] = acc

    for d, nsteps, rows in ((d0, _STEPS[0], _BLK), (d1, _STEPS[1], _BLK),
                            (d2, _STEPS[2], _DROWS[2])):
        @pl.when(step < nsteps)
        def _dense(d=d, rows=rows):
            lane = lax.broadcasted_iota(jnp.int32, (rows, 128), 1)
            confm = (lane % 8) == 4
            out_ref[0, 0] += 0.5 * jnp.sum(jnp.where(confm, _bce0(d[...]), 0.0))

    @pl.when(step == _STEPS[0] - 1)
    def _scale():
        out_ref[0, 0] = out_ref[0, 0] / float(B)


_final = pl.pallas_call(
    _final_body,
    grid=(_STEPS[0],),
    in_specs=[
        pl.BlockSpec((_BLK, 128), lambda i: (i, 0)),
        pl.BlockSpec((_BLK, 128), lambda i: (jnp.minimum(i, _STEPS[1] - 1), 0)),
        pl.BlockSpec((_DROWS[2], 128), lambda i: (0, 0)),
        pl.BlockSpec((128, 30, 128), lambda i: (0, 0, 0)),
        pl.BlockSpec((128, 30, 128), lambda i: (0, 0, 0)),
        pl.BlockSpec((128, 30, 128), lambda i: (0, 0, 0)),
        pl.BlockSpec((8, 30, 128), lambda i: (0, 0, 0)),
        pl.BlockSpec((8, 30, 128), lambda i: (0, 0, 0)),
        pl.BlockSpec((8, 30, 128), lambda i: (0, 0, 0)),
    ],
    out_specs=pl.BlockSpec((1, 1), lambda i: (0, 0), memory_space=pltpu.SMEM),
    out_shape=jax.ShapeDtypeStruct((1, 1), jnp.float32),
)


def kernel(out0, out1, out2, boxes, labels, iscrowd, area,
           anchors0, anchors1, anchors2):
    del iscrowd, area, anchors0, anchors1, anchors2
    i0, p0, i1, p1, i2, p2 = _prep(boxes, labels.astype(jnp.int32))
    t0 = out0.reshape(-1, 128)
    t1 = out1.reshape(-1, 128)
    t2 = out2.reshape(-1, 128)
    g0, g1, g2 = _make_sc_gather()(t0, t1, t2,
                            i0.reshape(NSLOT), i1.reshape(NSLOT), i2.reshape(NSLOT))
    pts = [g.T.reshape(128, 30, 128) for g in (g0, g1, g2)]
    pks = [p.reshape(8, 30, 128) for p in (p0, p1, p2)]
    d0 = out0.reshape(_DROWS[0], 128)
    d1 = out1.reshape(_DROWS[1], 128)
    d2 = out2.reshape(_DROWS[2], 128)
    res = _final(d0, d1, d2, *pts, *pks)
    return res.reshape(())


# speedup vs baseline: 2.3881x; 2.3881x over previous
import functools

import numpy as np
import jax
import jax.numpy as jnp
from jax import lax
from jax.experimental import pallas as pl
from jax.experimental.pallas import tpu as pltpu
from jax.experimental.pallas import tpu_sc as plsc

B = 64
NOBJ = 20
A = 3
NCLS = 3
CH = 8
GS = (76, 38, 19)
NS = tuple(A * g * g for g in GS)
IGNORE = 0.5
NSLOT = A * B * NOBJ
NW = 32
GATH_W = NSLOT * CH // NW
CHUNK = 120

_ANCH = (
    tuple((w / 8.0, h / 8.0) for w, h in ((10., 13.), (16., 30.), (33., 23.))),
    tuple((w / 16.0, h / 16.0) for w, h in ((30., 61.), (62., 45.), (59., 119.))),
    tuple((w / 32.0, h / 32.0) for w, h in ((116., 90.), (156., 198.), (373., 326.))),
)

_DROWS = (69312, 17328, 4332)
_BLK0 = 8664
_BLK1 = 2888
_NSTEP = 8


def _bce0(p):
    return -jnp.maximum(jnp.log(1.0 - p), -100.0)


def _bce1(p):
    return -jnp.maximum(jnp.log(p), -100.0)


def _prep_body(boxes_ref, labels_ref, i0, p0, i1, p1, i2, p2):
    bx = boxes_ref[...]
    lbl = labels_ref[...]
    bidx = lax.broadcasted_iota(jnp.int32, (B, NOBJ), 0)
    n1 = lax.broadcasted_iota(jnp.int32, (B, NOBJ, NOBJ), 1)
    n2 = lax.broadcasted_iota(jnp.int32, (B, NOBJ, NOBJ), 2)
    later = n2 > n1
    earlier = n2 < n1
    for s, (iref, pref) in enumerate(((i0, p0), (i1, p1), (i2, p2))):
        g = float(GS[s])
        gn = GS[s]
        tx = bx[:, :, 0] * g
        ty = bx[:, :, 1] * g
        tw = bx[:, :, 2] * g
        th = bx[:, :, 3] * g
        gi = tx.astype(jnp.int32)
        gj = ty.astype(jnp.int32)
        ious = []
        for a in range(A):
            aw, ah = _ANCH[s][a]
            inter = jnp.minimum(aw, tw) * jnp.minimum(ah, th)
            union = aw * ah + 1e-16 + tw * th - inter
            ious.append(inter / union)
        b01 = jnp.where(ious[1] > ious[0], 1, 0)
        m01 = jnp.maximum(ious[0], ious[1])
        best = jnp.where(ious[2] > m01, 2, b01)
        same = (gi[:, :, None] == gi[:, None, :]) & (gj[:, :, None] == gj[:, None, :])
        dup = same & (best[:, :, None] == best[:, None, :])
        winner = jnp.logical_not(jnp.any(dup & later, axis=2))
        tcks = []
        for k in range(NCLS):
            tck = jnp.any(dup & (lbl[:, None, :] == k), axis=2)
            tcks.append(tck.astype(jnp.float32))
        for a in range(A):
            z_a = (best == a) | (ious[a] > IGNORE)
            repz = z_a & jnp.logical_not(
                jnp.any(same & z_a[:, None, :] & earlier, axis=2))
            cell = (a * gn + gj) * gn + gi
            for c in range(CH):
                iref[c, a] = (bidx * CH + c) * NS[s] + cell
            pref[0, a] = repz.astype(jnp.float32)
            pref[1, a] = (winner & (best == a)).astype(jnp.float32)
            pref[2, a] = tw
            pref[3, a] = th
            for k in range(NCLS):
                pref[4 + k, a] = tcks[k]


_prep = pl.pallas_call(
    _prep_body,
    out_shape=(
        jax.ShapeDtypeStruct((CH, A, B, NOBJ), jnp.int32),
        jax.ShapeDtypeStruct((7, A, B, NOBJ), jnp.float32),
        jax.ShapeDtypeStruct((CH, A, B, NOBJ), jnp.int32),
        jax.ShapeDtypeStruct((7, A, B, NOBJ), jnp.float32),
        jax.ShapeDtypeStruct((CH, A, B, NOBJ), jnp.int32),
        jax.ShapeDtypeStruct((7, A, B, NOBJ), jnp.float32),
    ),
)


@functools.cache
def _make_sc_gather():
    mesh = plsc.VectorSubcoreMesh(core_axis_name="c", subcore_axis_name="s")

    @functools.partial(
        pl.kernel,
        mesh=mesh,
        out_type=[jax.ShapeDtypeStruct((NSLOT * CH,), jnp.float32)] * 3,
        scratch_types=[
            pltpu.VMEM((CHUNK,), jnp.int32),
            pltpu.VMEM((GATH_W,), jnp.float32),
            pltpu.SemaphoreType.DMA,
        ],
    )
    def _sc_gather(t0, t1, t2, i0, i1, i2, o0, o1, o2, idx_v, val_v, sem):
        wid = lax.axis_index("s") * 2 + lax.axis_index("c")
        base = wid * GATH_W
        for t, i, o in ((t0, i0, o0), (t1, i1, o1), (t2, i2, o2)):
            for k in range(GATH_W // CHUNK):
                pltpu.sync_copy(i.at[pl.ds(base + k * CHUNK, CHUNK)], idx_v)
                pltpu.async_copy(
                    t.at[idx_v], val_v.at[pl.ds(k * CHUNK, CHUNK)], sem
                ).wait()
            pltpu.sync_copy(val_v, o.at[pl.ds(base, GATH_W)])

    return _sc_gather


def _final_body(d0, d1, d2, pt0, pt1, pt2, pk0, pk1, pk2, out_ref):
    step = pl.program_id(0)

    @pl.when(step == 0)
    def _sparse():
        acc = jnp.float32(0.0)
        r = lax.broadcasted_iota(jnp.int32, (30, 128), 0) // 10
        for s, (pt, pk) in enumerate(((pt0, pk0), (pt1, pk1), (pt2, pk2))):
            a0, a1, a2 = _ANCH[s]
            aw = jnp.where(r == 0, a0[0], jnp.where(r == 1, a1[0], a2[0]))
            ah = jnp.where(r == 0, a0[1], jnp.where(r == 1, a1[1], a2[1]))
            px = pt[0]
            py = pt[1]
            pw = pt[2]
            ph = pt[3]
            pc = pt[4]
            zm = pk[0]
            om = pk[1]
            tw = pk[2]
            th = pk[3]
            fx = px - jnp.floor(px)
            fy = py - jnp.floor(py)
            dw = jnp.log(pw / aw + 1e-16) - jnp.log(tw / aw + 1e-16)
            dh = jnp.log(ph / ah + 1e-16) - jnp.log(th / ah + 1e-16)
            cls = jnp.float32(0.0)
            for k in range(NCLS):
                pkk = pt[5 + k]
                tck = pk[4 + k]
                cls = cls + tck * _bce1(pkk) + (1.0 - tck) * _bce0(pkk)
            objterm = fx * fx + fy * fy + dw * dw + dh * dh + 5.0 * _bce1(pc) + cls
            acc = acc + jnp.sum(om * objterm) - 0.5 * jnp.sum(zm * _bce0(pc))
        out_ref[0, 0] = acc

    def _dense(d, rows, s, base):
        per_img = float(CH * NS[s])
        pos = (lax.broadcasted_iota(jnp.int32, (rows, 128), 0) * 128
               + lax.broadcasted_iota(jnp.int32, (rows, 128), 1) + base)
        posf = pos.astype(jnp.float32)
        img = jnp.floor((posf + 0.5) * (1.0 / per_img))
        pin = posf - img * per_img
        confm = (pin >= float(4 * NS[s])) & (pin < float(5 * NS[s]))
        x = jnp.where(confm, 1.0 - d[...], 1.0)
        out_ref[0, 0] += -0.5 * jnp.sum(jnp.maximum(jnp.log(x), -100.0))

    _dense(d0, _BLK0, 0, step * (_BLK0 * 128))

    @pl.when(step < 6)
    def _dense1():
        _dense(d1, _BLK1, 1, step * (_BLK1 * 128))

    @pl.when(step == 0)
    def _dense2():
        _dense(d2, _DROWS[2], 2, 0)

    @pl.when(step == _NSTEP - 1)
    def _scale():
        out_ref[0, 0] = out_ref[0, 0] / float(B)


_final = pl.pallas_call(
    _final_body,
    grid=(_NSTEP,),
    in_specs=[
        pl.BlockSpec((_BLK0, 128), lambda i: (i, 0)),
        pl.BlockSpec((_BLK1, 128), lambda i: (jnp.minimum(i, 5), 0)),
        pl.BlockSpec((_DROWS[2], 128), lambda i: (0, 0)),
        pl.BlockSpec((CH, 30, 128), lambda i: (0, 0, 0)),
        pl.BlockSpec((CH, 30, 128), lambda i: (0, 0, 0)),
        pl.BlockSpec((CH, 30, 128), lambda i: (0, 0, 0)),
        pl.BlockSpec((7, 30, 128), lambda i: (0, 0, 0)),
        pl.BlockSpec((7, 30, 128), lambda i: (0, 0, 0)),
        pl.BlockSpec((7, 30, 128), lambda i: (0, 0, 0)),
    ],
    out_specs=pl.BlockSpec((1, 1), lambda i: (0, 0), memory_space=pltpu.SMEM),
    out_shape=jax.ShapeDtypeStruct((1, 1), jnp.float32),
)


def kernel(out0, out1, out2, boxes, labels, iscrowd, area,
           anchors0, anchors1, anchors2):
    del iscrowd, area, anchors0, anchors1, anchors2
    i0, p0, i1, p1, i2, p2 = _prep(boxes, labels.astype(jnp.int32))
    f0, f1, f2 = (jnp.transpose(o, (0, 2, 1)).reshape(-1)
                  for o in (out0, out1, out2))
    g0, g1, g2 = _make_sc_gather()(
        f0, f1, f2, i0.reshape(-1), i1.reshape(-1), i2.reshape(-1))
    pts = [g.reshape(CH, 30, 128) for g in (g0, g1, g2)]
    pks = [p.reshape(7, 30, 128) for p in (p0, p1, p2)]
    d0 = f0.reshape(_DROWS[0], 128)
    d1 = f1.reshape(_DROWS[1], 128)
    d2 = f2.reshape(_DROWS[2], 128)
    res = _final(d0, d1, d2, *pts, *pks)
    return res.reshape(())
